# baseline (speedup 1.0000x reference)
import functools
import math

import jax
import jax.numpy as jnp
from jax import lax
from jax.experimental import pallas as pl
from jax.experimental.pallas import tpu as pltpu
from jax.experimental.pallas import tpu_sc as plsc

_NC = 2
_NS = 16
_NW = _NC * _NS

_CHUNK = 128
_NBUF = 6
_AHEAD = _NBUF - 2
_L = 16
_RPI = 4


@functools.lru_cache(maxsize=None)
def _make_gather(B, D):
    assert B % (8 * _NW) == 0
    b_per_w = B // _NW
    assert b_per_w % _CHUNK == 0
    steps = b_per_w // _CHUNK
    assert steps >= 3 * _NBUF
    assert _CHUNK % _RPI == 0 and D % _L == 0
    scale = math.sqrt(float(D))
    mesh = plsc.VectorSubcoreMesh(core_axis_name="c", subcore_axis_name="s")

    @functools.partial(
        pl.kernel,
        mesh=mesh,
        out_type=jax.ShapeDtypeStruct((B, D), jnp.float32),
        scratch_types=[
            pltpu.VMEM((b_per_w,), jnp.int32),
        ] + [pltpu.VMEM((_CHUNK, D), jnp.float32)] * _NBUF
          + [pltpu.SemaphoreType.DMA] * (2 * _NBUF),
    )
    def gather_kernel(table_hbm, idx_hbm, out_hbm, idx_v, *bufs_and_sems):
        rows = bufs_and_sems[:_NBUF]
        gs = bufs_and_sems[_NBUF:2 * _NBUF]
        sts = bufs_and_sems[2 * _NBUF:]
        wid = lax.axis_index("s") * _NC + lax.axis_index("c")
        base = wid * b_per_w

        def start_gather(j, u):
            pltpu.async_copy(
                table_hbm.at[idx_v.at[pl.ds(j * _CHUNK, _CHUNK)]], rows[u], gs[u])

        def wait_gather(u):
            pltpu.make_async_copy(
                table_hbm.at[idx_v.at[pl.ds(0, _CHUNK)]], rows[u], gs[u]).wait()

        def start_store(j, u):
            pltpu.async_copy(
                rows[u], out_hbm.at[pl.ds(base + j * _CHUNK, _CHUNK)], sts[u])

        def wait_store(u):
            pltpu.make_async_copy(
                rows[u], out_hbm.at[pl.ds(base, _CHUNK)], sts[u]).wait()

        def scale_buf(u):
            r = rows[u]

            def srow(i, carry):
                for rr in range(_RPI):
                    for c in range(D // _L):
                        sl = pl.ds(c * _L, _L)
                        r[i * _RPI + rr, sl] = r[i * _RPI + rr, sl] * scale
                return carry

            lax.fori_loop(0, _CHUNK // _RPI, srow, 0)

        def full(j, u, st_wait=True):
            bg = (u + _AHEAD) % _NBUF
            if st_wait:
                wait_store(bg)
            start_gather(j + _AHEAD, bg)
            wait_gather(u)
            scale_buf(u)
            start_store(j, u)

        def tail(j, u):
            wait_gather(u)
            scale_buf(u)
            start_store(j, u)

        pltpu.sync_copy(idx_hbm.at[pl.ds(base, b_per_w)], idx_v)
        for j in range(_AHEAD):
            start_gather(j, j)
        for j in range(_NBUF):
            full(j, j, st_wait=(j >= 2))

        def body(gp, carry):
            j0 = _NBUF * gp + _NBUF
            for u in range(_NBUF):
                full(j0 + u, u)
            return carry

        ngroups = (steps - _NBUF - _AHEAD) // _NBUF
        lax.fori_loop(0, ngroups, body, 0)

        for j in range(_NBUF * (ngroups + 1), steps):
            if j + _AHEAD < steps:
                full(j, j % _NBUF)
            else:
                tail(j, j % _NBUF)
        for u in range(_NBUF):
            wait_store(u)

    return gather_kernel


def kernel(x, table):
    n, s = x.shape
    v, d = table.shape
    b = n * s
    idx = x.reshape(b).astype(jnp.int32)
    out = _make_gather(b, d)(table, idx)
    return out.reshape(n, s, d)


# speedup vs baseline: 1.0003x; 1.0003x over previous
import functools
import math

import jax
import jax.numpy as jnp
from jax import lax
from jax.experimental import pallas as pl
from jax.experimental.pallas import tpu as pltpu
from jax.experimental.pallas import tpu_sc as plsc

_NC = 2
_NS = 16
_NW = _NC * _NS

_CHUNK = 200
_NBUF = 4
_AHEAD = _NBUF - 2
_L = 16
_RPI = 4


@functools.lru_cache(maxsize=None)
def _make_gather(B, D):
    assert B % (8 * _NW) == 0
    b_per_w = B // _NW
    assert b_per_w % _CHUNK == 0
    steps = b_per_w // _CHUNK
    assert steps % _NBUF == 0 and steps >= 3 * _NBUF
    assert _CHUNK % _RPI == 0 and D % _L == 0
    scale = math.sqrt(float(D))
    mesh = plsc.VectorSubcoreMesh(core_axis_name="c", subcore_axis_name="s")

    @functools.partial(
        pl.kernel,
        mesh=mesh,
        out_type=jax.ShapeDtypeStruct((B, D), jnp.float32),
        scratch_types=[
            pltpu.VMEM((b_per_w,), jnp.int32),
        ] + [pltpu.VMEM((_CHUNK, D), jnp.float32)] * _NBUF
          + [pltpu.SemaphoreType.DMA] * (2 * _NBUF),
    )
    def gather_kernel(table_hbm, idx_hbm, out_hbm, idx_v, *bufs_and_sems):
        rows = bufs_and_sems[:_NBUF]
        gs = bufs_and_sems[_NBUF:2 * _NBUF]
        sts = bufs_and_sems[2 * _NBUF:]
        wid = lax.axis_index("s") * _NC + lax.axis_index("c")
        base = wid * b_per_w

        def start_gather(j, u):
            pltpu.async_copy(
                table_hbm.at[idx_v.at[pl.ds(j * _CHUNK, _CHUNK)]], rows[u], gs[u])

        def wait_gather(u):
            pltpu.make_async_copy(
                table_hbm.at[idx_v.at[pl.ds(0, _CHUNK)]], rows[u], gs[u]).wait()

        def start_store(j, u):
            pltpu.async_copy(
                rows[u], out_hbm.at[pl.ds(base + j * _CHUNK, _CHUNK)], sts[u])

        def wait_store(u):
            pltpu.make_async_copy(
                rows[u], out_hbm.at[pl.ds(base, _CHUNK)], sts[u]).wait()

        def scale_buf(u):
            r = rows[u]

            def srow(i, carry):
                for rr in range(_RPI):
                    for c in range(D // _L):
                        sl = pl.ds(c * _L, _L)
                        r[i * _RPI + rr, sl] = r[i * _RPI + rr, sl] * scale
                return carry

            lax.fori_loop(0, _CHUNK // _RPI, srow, 0)

        def full(j, u, st_wait=True):
            bg = (u + _AHEAD) % _NBUF
            if st_wait:
                wait_store(bg)
            start_gather(j + _AHEAD, bg)
            wait_gather(u)
            scale_buf(u)
            start_store(j, u)

        def tail(j, u):
            wait_gather(u)
            scale_buf(u)
            start_store(j, u)

        pltpu.sync_copy(idx_hbm.at[pl.ds(base, b_per_w)], idx_v)
        for j in range(_AHEAD):
            start_gather(j, j)
        for j in range(_NBUF):
            full(j, j, st_wait=(j >= 2))

        def body(gp, carry):
            j0 = _NBUF * gp + _NBUF
            for u in range(_NBUF):
                full(j0 + u, u)
            return carry

        ngroups = (steps - _NBUF - _AHEAD) // _NBUF
        lax.fori_loop(0, ngroups, body, 0)

        for j in range(_NBUF * (ngroups + 1), steps):
            if j + _AHEAD < steps:
                full(j, j % _NBUF)
            else:
                tail(j, j % _NBUF)
        for u in range(_NBUF):
            wait_store(u)

    return gather_kernel


def kernel(x, table):
    n, s = x.shape
    v, d = table.shape
    b = n * s
    idx = x.reshape(b).astype(jnp.int32)
    out = _make_gather(b, d)(table, idx)
    return out.reshape(n, s, d)
